# baseline (speedup 1.0000x reference)
import functools

import jax
import jax.numpy as jnp
from jax import lax
from jax.experimental import pallas as pl
from jax.experimental.pallas import tpu as pltpu
from jax.experimental.pallas import tpu_sc as plsc

DIM = 64
NC, NS = 2, 16
NW = NC * NS
BBLK = 128
NBUF = 2
K = 1
VPAD = 1024
L = 16


@functools.cache
def _emb_call(batch, seq):
    n_chunks = seq
    n_outer = n_chunks // NBUF
    assert batch == NW * BBLK and n_chunks % NBUF == 0 and n_outer >= 2
    mesh = plsc.VectorSubcoreMesh(core_axis_name="c", subcore_axis_name="s")

    @functools.partial(
        pl.kernel,
        out_type=jax.ShapeDtypeStruct((seq, DIM, batch), jnp.float32),
        mesh=mesh,
        scratch_types=(
            [pltpu.VMEM((seq, BBLK), jnp.int32),
             pltpu.VMEM((NBUF, BBLK, 128), jnp.float32),
             pltpu.VMEM((NBUF, DIM, BBLK), jnp.float32),
             pltpu.VMEM_SHARED((VPAD, 128), jnp.float32),
             pltpu.VMEM((16, 128), jnp.float32)]
            + [pltpu.SemaphoreType.DMA] * (2 * NBUF)
        ),
        compiler_params=pltpu.CompilerParams(use_tc_tiling_on_sc=True,
                                             needs_layout_passes=False),
    )
    def emb(idx_hbm, table_hbm, out_hbm, idx_v, rows_v, blk_v, table_sp,
            peek_v, *sems):
        sem_g, sem_s = sems[:NBUF], sems[NBUF:]
        sid = lax.axis_index("s")
        wid = sid * NC + lax.axis_index("c")
        pltpu.sync_copy(table_hbm.at[pl.ds(sid * 64, 64)],
                        table_sp.at[pl.ds(sid * 64, 64)])
        pltpu.sync_copy(idx_hbm.at[slice(None), pl.ds(wid * BBLK, BBLK)],
                        idx_v)
        plsc.subcore_barrier()
        nb = lax.rem(sid + 1, 16)
        pltpu.sync_copy(table_sp.at[pl.ds(nb * 64 + 48, 16)], peek_v)
        plsc.subcore_barrier()

        def gather(j, b):
            pltpu.async_copy(table_sp.at[idx_v.at[j]], rows_v.at[b],
                             sem_g[b])

        def gather_wait(j, b):
            pltpu.make_async_copy(table_sp.at[idx_v.at[j]], rows_v.at[b],
                                  sem_g[b]).wait()

        def store(j, b):
            pltpu.async_copy(blk_v.at[b],
                             out_hbm.at[j, slice(None),
                                        pl.ds(wid * BBLK, BBLK)],
                             sem_s[b])

        def store_wait(j, b):
            pltpu.make_async_copy(blk_v.at[b],
                                  out_hbm.at[j, slice(None),
                                             pl.ds(wid * BBLK, BBLK)],
                                  sem_s[b]).wait()

        iota = lax.iota(jnp.int32, L)
        r_base = [iota + (L * i) for i in range(BBLK // L)]

        def transpose(b):
            rows = rows_v.at[b]
            for d in range(DIM):
                c = jnp.full((L,), d, jnp.int32)
                for i in range(BBLK // L):
                    v = plsc.load_gather(rows, [r_base[i], c])
                    blk_v[b, d, pl.ds(L * i, L)] = v

        for b in range(K):
            gather(b, b)

        for b in range(NBUF):
            bb = (b + K) % NBUF
            if b >= NBUF - K:
                store_wait(b - (NBUF - K), bb)
            gather(b + K, bb)
            gather_wait(b, b)
            transpose(b)
            store(b, b)

        def outer(o, carry):
            j0 = o * NBUF
            for b in range(NBUF):
                j = j0 + b
                bb = (b + K) % NBUF
                store_wait(j - (NBUF - K), bb)
                gather(j + K, bb)
                gather_wait(j, b)
                transpose(b)
                store(j, b)
            return carry

        lax.fori_loop(1, n_outer - 1, outer, 0)

        j0 = (n_outer - 1) * NBUF
        for b in range(NBUF):
            j = j0 + b
            bb = (b + K) % NBUF
            if b < NBUF - K:
                store_wait(j - (NBUF - K), bb)
                gather(j + K, bb)
            gather_wait(j, b)
            transpose(b)
            store(j, b)

        for b in range(NBUF):
            store_wait(j0 + b, b)

    return emb


def kernel(x, table):
    b, s = x.shape
    idx = x.T.astype(jnp.int32)
    tab = jnp.pad(table.astype(jnp.float32),
                  ((0, VPAD - table.shape[0]), (0, 128 - table.shape[1])))
    out = _emb_call(b, s)(idx, tab)
    return jnp.transpose(out, (2, 0, 1))


# speedup vs baseline: 2.1494x; 2.1494x over previous
import functools

import jax
import jax.numpy as jnp
from jax import lax
from jax.experimental import pallas as pl
from jax.experimental.pallas import tpu as pltpu
from jax.experimental.pallas import tpu_sc as plsc

DIM = 64
NC, NS = 2, 16
NW = NC * NS
BBLK = 128
NBUF = 2
K = 1
VPAD = 1024
L = 16


@functools.cache
def _emb_call(batch, seq):
    n_chunks = seq
    n_outer = n_chunks // NBUF
    assert batch == NW * BBLK and n_chunks % NBUF == 0 and n_outer >= 2
    mesh = plsc.VectorSubcoreMesh(core_axis_name="c", subcore_axis_name="s")

    @functools.partial(
        pl.kernel,
        out_type=jax.ShapeDtypeStruct((seq, DIM, batch), jnp.float32),
        mesh=mesh,
        scratch_types=(
            [pltpu.VMEM((seq, BBLK), jnp.int32),
             pltpu.VMEM((NBUF, BBLK, 128), jnp.float32),
             pltpu.VMEM((NBUF, DIM, BBLK), jnp.float32),
             pltpu.VMEM_SHARED((VPAD, 128), jnp.float32),
             pltpu.VMEM((16, 128), jnp.float32)]
            + [pltpu.SemaphoreType.DMA] * (2 * NBUF)
        ),
        compiler_params=pltpu.CompilerParams(use_tc_tiling_on_sc=True,
                                             needs_layout_passes=False),
    )
    def emb(idx_hbm, table_hbm, out_hbm, idx_v, rows_v, blk_v, table_sp,
            peek_v, *sems):
        sem_g, sem_s = sems[:NBUF], sems[NBUF:]
        sid = lax.axis_index("s")
        wid = sid * NC + lax.axis_index("c")
        pltpu.sync_copy(table_hbm.at[pl.ds(sid * 64, 64)],
                        table_sp.at[pl.ds(sid * 64, 64)])
        pltpu.sync_copy(idx_hbm.at[slice(None), pl.ds(wid * BBLK, BBLK)],
                        idx_v)
        plsc.subcore_barrier()
        nb = lax.rem(sid + 1, 16)
        pltpu.sync_copy(table_sp.at[pl.ds(nb * 64 + 48, 16)], peek_v)
        plsc.subcore_barrier()

        def gather(j, b):
            pltpu.async_copy(table_sp.at[idx_v.at[j]], rows_v.at[b],
                             sem_g[b])

        def gather_wait(j, b):
            pltpu.make_async_copy(table_sp.at[idx_v.at[j]], rows_v.at[b],
                                  sem_g[b]).wait()

        def store(j, b):
            pltpu.async_copy(blk_v.at[b],
                             out_hbm.at[j, slice(None),
                                        pl.ds(wid * BBLK, BBLK)],
                             sem_s[b])

        def store_wait(j, b):
            pltpu.make_async_copy(blk_v.at[b],
                                  out_hbm.at[j, slice(None),
                                             pl.ds(wid * BBLK, BBLK)],
                                  sem_s[b]).wait()

        iota = lax.iota(jnp.int32, L)
        r_base = [iota + (L * i) for i in range(BBLK // L)]

        def transpose(b):
            rows = rows_v.at[b]

            @plsc.parallel_loop(0, DIM, 1, unroll=8)
            def _(d):
                c = jnp.zeros((L,), jnp.int32) + d
                for i in range(BBLK // L):
                    v = plsc.load_gather(rows, [r_base[i], c])
                    blk_v[b, d, pl.ds(L * i, L)] = v

        for b in range(K):
            gather(b, b)

        for b in range(NBUF):
            bb = (b + K) % NBUF
            if b >= NBUF - K:
                store_wait(b - (NBUF - K), bb)
            gather(b + K, bb)
            gather_wait(b, b)
            transpose(b)
            store(b, b)

        def outer(o, carry):
            j0 = o * NBUF
            for b in range(NBUF):
                j = j0 + b
                bb = (b + K) % NBUF
                store_wait(j - (NBUF - K), bb)
                gather(j + K, bb)
                gather_wait(j, b)
                transpose(b)
                store(j, b)
            return carry

        lax.fori_loop(1, n_outer - 1, outer, 0)

        j0 = (n_outer - 1) * NBUF
        for b in range(NBUF):
            j = j0 + b
            bb = (b + K) % NBUF
            if b < NBUF - K:
                store_wait(j - (NBUF - K), bb)
                gather(j + K, bb)
            gather_wait(j, b)
            transpose(b)
            store(j, b)

        for b in range(NBUF):
            store_wait(j0 + b, b)

    return emb


def kernel(x, table):
    b, s = x.shape
    idx = x.T.astype(jnp.int32)
    tab = jnp.pad(table.astype(jnp.float32),
                  ((0, VPAD - table.shape[0]), (0, 128 - table.shape[1])))
    out = _emb_call(b, s)(idx, tab)
    return jnp.transpose(out, (2, 0, 1))


# speedup vs baseline: 3.4539x; 1.6070x over previous
import functools

import jax
import jax.numpy as jnp
from jax import lax
from jax.experimental import pallas as pl
from jax.experimental.pallas import tpu as pltpu
from jax.experimental.pallas import tpu_sc as plsc

DIM = 64
NC, NS = 2, 16
NW = NC * NS
CHUNK = 200
HALF = CHUNK // 2
NBUF = 2
K = 1
VPAD = 1024


@functools.cache
def _emb_call(batch, seq):
    n_total = batch * seq
    b_per_w = batch // NW
    n_chunks = b_per_w
    n_outer = n_chunks // NBUF
    assert seq == CHUNK and n_chunks % NBUF == 0 and n_outer >= 2
    mesh = plsc.VectorSubcoreMesh(core_axis_name="c", subcore_axis_name="s")

    @functools.partial(
        pl.kernel,
        out_type=jax.ShapeDtypeStruct((batch, seq, DIM), jnp.float32),
        mesh=mesh,
        scratch_types=(
            [pltpu.VMEM((2 * n_chunks, HALF), jnp.int32),
             pltpu.VMEM((NBUF, CHUNK, DIM), jnp.float32),
             pltpu.VMEM_SHARED((VPAD, DIM), jnp.float32),
             pltpu.VMEM((16, DIM), jnp.float32)]
            + [pltpu.SemaphoreType.DMA] * (2 * NBUF)
        ),
        compiler_params=pltpu.CompilerParams(use_tc_tiling_on_sc=True),
    )
    def emb(idx_hbm, table_hbm, out_hbm, idx_v, rows_v, table_sp, peek_v,
            *sems):
        sem_g, sem_s = sems[:NBUF], sems[NBUF:]
        sid = lax.axis_index("s")
        wid = sid * NC + lax.axis_index("c")
        pltpu.sync_copy(table_hbm.at[pl.ds(sid * 64, 64)],
                        table_sp.at[pl.ds(sid * 64, 64)])
        pltpu.sync_copy(idx_hbm.at[pl.ds(wid * 2 * n_chunks, 2 * n_chunks)],
                        idx_v)
        plsc.subcore_barrier()
        nb = lax.rem(sid + 1, 16)
        pltpu.sync_copy(table_sp.at[pl.ds(nb * 64 + 48, 16)], peek_v)
        plsc.subcore_barrier()
        base_b = wid * b_per_w

        def gather(j, b):
            pltpu.async_copy(table_sp.at[idx_v.at[2 * j]],
                             rows_v.at[b, pl.ds(0, HALF)], sem_g[b])
            pltpu.async_copy(table_sp.at[idx_v.at[2 * j + 1]],
                             rows_v.at[b, pl.ds(HALF, HALF)], sem_g[b])

        def gather_wait(j, b):
            pltpu.make_async_copy(table_sp.at[idx_v.at[2 * j]],
                                  rows_v.at[b, pl.ds(0, HALF)],
                                  sem_g[b]).wait()
            pltpu.make_async_copy(table_sp.at[idx_v.at[2 * j + 1]],
                                  rows_v.at[b, pl.ds(HALF, HALF)],
                                  sem_g[b]).wait()

        def store(j, b):
            pltpu.async_copy(rows_v.at[b], out_hbm.at[base_b + j], sem_s[b])

        def store_wait(j, b):
            pltpu.make_async_copy(rows_v.at[b], out_hbm.at[base_b + j],
                                  sem_s[b]).wait()

        for b in range(K):
            gather(b, b)

        for b in range(NBUF):
            bb = (b + K) % NBUF
            if b >= NBUF - K:
                store_wait(b - (NBUF - K), bb)
            gather(b + K, bb)
            gather_wait(b, b)
            store(b, b)

        def outer(o, carry):
            j0 = o * NBUF
            for b in range(NBUF):
                j = j0 + b
                bb = (b + K) % NBUF
                store_wait(j - (NBUF - K), bb)
                gather(j + K, bb)
                gather_wait(j, b)
                store(j, b)
            return carry

        lax.fori_loop(1, n_outer - 1, outer, 0)

        j0 = (n_outer - 1) * NBUF
        for b in range(NBUF):
            j = j0 + b
            bb = (b + K) % NBUF
            if b < NBUF - K:
                store_wait(j - (NBUF - K), bb)
                gather(j + K, bb)
            gather_wait(j, b)
            store(j, b)

        for b in range(NBUF):
            store_wait(j0 + b, b)

    return emb


def kernel(x, table):
    b, s = x.shape
    idx = x.reshape(2 * b, HALF).astype(jnp.int32)
    tab = jnp.pad(table.astype(jnp.float32),
                  ((0, VPAD - table.shape[0]), (0, 0)))
    return _emb_call(b, s)(idx, tab)
